# baseline (speedup 1.0000x reference)
import functools

import jax
import jax.numpy as jnp
from jax import lax
from jax.experimental import pallas as pl
from jax.experimental.pallas import tpu as pltpu
from jax.experimental.pallas import tpu_sc as plsc

B = 32
V = 100000
K = 64
L = 16
NCHUNK = V // L
NBKT1 = 8192
NBKT2 = 2048
NBKT3 = 256
CAP = 512
TEMP = 0.7
TOP_P = 0.95
NEG_INF = float("-inf")


def _iota():
    return lax.iota(jnp.int32, L)


def _extract(vec, j):
    return jnp.sum(jnp.where(_iota() == j, vec, jnp.zeros_like(vec)))


def _scalar0(vec):
    return _extract(vec, 0)


def _key16(x):
    bits = plsc.bitcast(x, jnp.int32)
    return jnp.where(bits < 0, bits ^ jnp.int32(0x7FFFFFFF), bits)


def _zero_hist(hist, nvregs):
    def zbody(j):
        hist[pl.ds(j * L, L)] = jnp.zeros((L,), jnp.int32)

    plsc.parallel_loop(0, nvregs, unroll=8)(zbody)


def _scan_top(hist, v_start, acc0, target):
    def cond(st):
        v, _, found, _, _ = st
        return jnp.logical_and(jnp.logical_not(found), v >= 0)

    def body(st):
        v, acc, _, b, ca = st
        h = hist[pl.ds(v * L, L)]
        srev = plsc.cumsum(lax.rev(h, (0,)))
        m = srev >= (target - acc)
        has = _scalar0(plsc.all_reduce_population_count(m)) > 0
        j0 = _scalar0(plsc.all_reduce_ffs(m))
        bl = (L - 1) - j0
        s_j0 = _extract(srev, j0)
        h_bl = _extract(h, bl)
        tot = _extract(srev, L - 1)
        b_new = v * L + bl
        ca_new = acc + s_j0 - h_bl
        return (
            v - 1,
            jnp.where(has, acc, acc + tot),
            has,
            jnp.where(has, b_new, b),
            jnp.where(has, ca_new, ca),
        )

    _, _, _, b, ca = lax.while_loop(
        cond, body, (v_start, acc0, False, jnp.int32(0), jnp.int32(0))
    )
    return b, ca


def _body(logits_hbm, probs_out, idx_out, data, hist, cvals, cidx,
          selx, oprob, oidx, sem0, sem1, sem2, sem3):
    nc = plsc.get_sparse_core_info().num_cores
    wid = lax.axis_index("s") * nc + lax.axis_index("c")

    row_copy = pltpu.async_copy(logits_hbm.at[wid], data, sem0)
    _zero_hist(hist, NBKT1 // L)
    row_copy.wait()

    def p1_body(i, mx):
        x = data[pl.ds(i * L, L)]
        key = _key16(x)
        bkt = (key >> 19) + jnp.int32(NBKT1 // 2)
        plsc.addupdate_scatter(hist, [bkt], jnp.ones((L,), jnp.int32))
        return jnp.maximum(mx, x)

    mx = plsc.parallel_loop(
        0, NCHUNK, unroll=16, carry=jnp.full((L,), NEG_INF, jnp.float32)
    )(p1_body)
    m_val = jnp.max(mx)
    m_y = jnp.full((L,), m_val) / jnp.full((L,), TEMP, jnp.float32)

    kmax = jnp.where(
        plsc.bitcast(jnp.full((L,), m_val), jnp.int32) < 0,
        plsc.bitcast(jnp.full((L,), m_val), jnp.int32) ^ jnp.int32(0x7FFFFFFF),
        plsc.bitcast(jnp.full((L,), m_val), jnp.int32),
    )
    kmax0 = _scalar0(kmax)
    v_start1 = ((kmax0 >> 19) + jnp.int32(NBKT1 // 2)) // L
    b1, ca1 = _scan_top(hist, v_start1, jnp.int32(0), jnp.int32(K))
    p1 = b1 - jnp.int32(NBKT1 // 2)
    n1 = ca1 + _extract(hist[pl.ds((b1 // L) * L, L)], b1 % L)

    def level23(_):
        _zero_hist(hist, NBKT2 // L)

        def h2_body(i, _):
            x = data[pl.ds(i * L, L)]
            key = _key16(x)
            sel = (key >> 19) == p1
            bkt = (key >> 8) & jnp.int32(0x7FF)
            plsc.addupdate_scatter(
                hist, [bkt], jnp.ones((L,), jnp.int32), mask=sel
            )
            return 0

        lax.fori_loop(0, NCHUNK, h2_body, 0)
        b2, ca2 = _scan_top(hist, jnp.int32(NBKT2 // L - 1), ca1, jnp.int32(K))
        n2 = ca2 + _extract(hist[pl.ds((b2 // L) * L, L)], b2 % L)
        t2 = (p1 << 19) | (b2 << 8)

        def level3(_):
            _zero_hist(hist, NBKT3 // L)
            pfx2 = (p1 << 11) | b2

            def h3_body(i, _):
                x = data[pl.ds(i * L, L)]
                key = _key16(x)
                sel = (key >> 8) == pfx2
                bkt = key & jnp.int32(0xFF)
                plsc.addupdate_scatter(
                    hist, [bkt], jnp.ones((L,), jnp.int32), mask=sel
                )
                return 0

            lax.fori_loop(0, NCHUNK, h3_body, 0)
            b3, ca3 = _scan_top(
                hist, jnp.int32(NBKT3 // L - 1), ca2, jnp.int32(K)
            )
            t_eq = t2 | b3
            return t_eq + jnp.int32(1), jnp.bool_(True), t_eq

        return lax.cond(
            n2 <= CAP,
            lambda _: (t2, jnp.bool_(False), jnp.int32(0)),
            level3,
            0,
        )

    thresh, need_eq, t_eq = lax.cond(
        n1 <= CAP,
        lambda _: (p1 << 19, jnp.bool_(False), jnp.int32(0)),
        level23,
        0,
    )

    tempv = jnp.full((L,), TEMP, jnp.float32)

    def p2_body(i, carry):
        ssum, cntv = carry
        x = data[pl.ds(i * L, L)]
        ssum = ssum + jnp.exp(x / tempv - m_y)
        key = _key16(x)
        m = key >= thresh
        inc = plsc.cumsum(jnp.where(m, jnp.int32(1), jnp.int32(0)))
        pos = cntv + inc - 1
        plsc.store_scatter(cvals, [pos], x, mask=m)
        plsc.store_scatter(cidx, [pos], i * L + _iota(), mask=m)
        cntv = cntv + plsc.all_reduce_population_count(m)
        return ssum, cntv

    ssum, cntv = plsc.parallel_loop(
        0, NCHUNK, unroll=16,
        carry=(jnp.zeros((L,), jnp.float32), jnp.zeros((L,), jnp.int32)),
    )(p2_body)
    s_val = jnp.sum(ssum)
    cnt = _scalar0(cntv)

    def eq_phase(cnt0):
        def cond(st):
            i, cnt = st
            return jnp.logical_and(i < NCHUNK, cnt < K)

        def body(st):
            i, cnt = st
            x = data[pl.ds(i * L, L)]
            key = _key16(x)
            m = key == t_eq
            inc = plsc.cumsum(jnp.where(m, jnp.int32(1), jnp.int32(0)))
            pos = cnt + inc - 1
            plsc.store_scatter(cvals, [pos], x, mask=m)
            plsc.store_scatter(cidx, [pos], i * L + _iota(), mask=m)
            cnt = cnt + _scalar0(plsc.all_reduce_population_count(m))
            return i + 1, cnt

        _, cnt1 = lax.while_loop(cond, body, (jnp.int32(0), cnt0))
        return cnt1

    cnt = lax.cond(need_eq, eq_phase, lambda c: c, cnt)

    nv = (cnt + L - 1) // L

    def pad_body(j, _):
        v = cvals[pl.ds(j * L, L)]
        posn = j * L + _iota()
        cvals[pl.ds(j * L, L)] = jnp.where(
            posn >= cnt, jnp.full((L,), NEG_INF, jnp.float32), v
        )
        return 0

    lax.fori_loop(0, nv, pad_body, 0)

    lane0 = _iota() == 0

    def sel_body(k, _):
        def scan_body(j, st):
            mvec, jvec = st
            v = cvals[pl.ds(j * L, L)]
            gt = v > mvec
            return jnp.where(gt, v, mvec), jnp.where(gt, j, jvec)

        mvec, jvec = lax.fori_loop(
            0, nv, scan_body,
            (jnp.full((L,), NEG_INF, jnp.float32), jnp.zeros((L,), jnp.int32)),
        )
        mval = jnp.max(mvec)
        posn = jvec * L + _iota()
        pos = jnp.min(
            jnp.where(mvec == mval, posn, jnp.int32(2 ** 30))
        )
        posv = jnp.zeros((L,), jnp.int32) + pos
        gidx = plsc.load_gather(cidx, [posv])
        plsc.store_scatter(
            cvals, [posv], jnp.full((L,), NEG_INF, jnp.float32), mask=lane0
        )
        kv = jnp.zeros((L,), jnp.int32) + k
        plsc.store_scatter(selx, [kv], jnp.full((L,), 0.0) + mval, mask=lane0)
        plsc.store_scatter(oidx, [kv], gidx, mask=lane0)
        return 0

    lax.fori_loop(0, K, sel_body, 0)

    carry = jnp.float32(0.0)
    psum = jnp.zeros((L,), jnp.float32)
    s_vec = jnp.zeros((L,), jnp.float32) + s_val
    for t in range(K // L):
        xv = selx[pl.ds(t * L, L)]
        p = jnp.exp(xv / tempv - m_y) / s_vec
        c = plsc.cumsum(p) + carry
        carry = _extract(c, L - 1)
        p = jnp.where((c - p) > TOP_P, jnp.float32(0.0), p)
        psum = psum + p
        oprob[pl.ds(t * L, L)] = p
    tot_vec = jnp.zeros((L,), jnp.float32) + jnp.sum(psum)
    for t in range(K // L):
        oprob[pl.ds(t * L, L)] = oprob[pl.ds(t * L, L)] / tot_vec

    pltpu.sync_copy(oprob, probs_out.at[pl.ds(wid * K, K)])
    pltpu.sync_copy(oidx, idx_out.at[pl.ds(wid * K, K)])


@jax.jit
def _sc_topk_sample(x2d):
    mesh = plsc.VectorSubcoreMesh(core_axis_name="c", subcore_axis_name="s")
    fn = pl.kernel(
        _body,
        out_type=[
            jax.ShapeDtypeStruct((B * K,), jnp.float32),
            jax.ShapeDtypeStruct((B * K,), jnp.int32),
        ],
        mesh=mesh,
        scratch_types=[
            pltpu.VMEM((V,), jnp.float32),
            pltpu.VMEM((NBKT1,), jnp.int32),
            pltpu.VMEM((CAP,), jnp.float32),
            pltpu.VMEM((CAP,), jnp.int32),
            pltpu.VMEM((K,), jnp.float32),
            pltpu.VMEM((K,), jnp.float32),
            pltpu.VMEM((K,), jnp.int32),
            pltpu.SemaphoreType.DMA,
            pltpu.SemaphoreType.DMA,
            pltpu.SemaphoreType.DMA,
            pltpu.SemaphoreType.DMA,
        ],
        name="sc_topk_sampler",
        compiler_params=pltpu.CompilerParams(
            needs_layout_passes=False, use_tc_tiling_on_sc=True
        ),
    )
    pr, ix = fn(x2d)
    return pr.reshape(B, K), ix.reshape(B, K)


def kernel(logits):
    x = logits[:, -1]
    probs_sorted, indices = _sc_topk_sample(x)
    skey = jax.random.fold_in(jax.random.key(42), 0)
    next_token = jax.random.categorical(skey, logits=jnp.log(probs_sorted))
    next_token = jnp.take_along_axis(indices, next_token[..., None], axis=-1)
    next_token = jnp.squeeze(next_token, axis=-1)
    return next_token, probs_sorted


# speedup vs baseline: 1.0510x; 1.0510x over previous
import functools

import jax
import jax.numpy as jnp
from jax import lax
from jax.experimental import pallas as pl
from jax.experimental.pallas import tpu as pltpu
from jax.experimental.pallas import tpu_sc as plsc

B = 32
V = 100000
K = 64
L = 16
NCHUNK = V // L
NBKT1 = 8192
NBKT2 = 2048
NBKT3 = 256
CAP = 512
TEMP = 0.7
TOP_P = 0.95
NEG_INF = float("-inf")


def _iota():
    return lax.iota(jnp.int32, L)


def _extract(vec, j):
    return jnp.sum(jnp.where(_iota() == j, vec, jnp.zeros_like(vec)))


def _scalar0(vec):
    return _extract(vec, 0)


def _key16(x):
    bits = plsc.bitcast(x, jnp.int32)
    return jnp.where(bits < 0, bits ^ jnp.int32(0x7FFFFFFF), bits)


def _zero_hist(hist, nvregs):
    def zbody(j):
        hist[pl.ds(j * L, L)] = jnp.zeros((L,), jnp.int32)

    plsc.parallel_loop(0, nvregs, unroll=8)(zbody)


def _scan_top(hist, v_start, acc0, target):
    def cond(st):
        v, _, found, _, _ = st
        return jnp.logical_and(jnp.logical_not(found), v >= 0)

    def body(st):
        v, acc, _, b, ca = st
        h = hist[pl.ds(v * L, L)]
        srev = plsc.cumsum(lax.rev(h, (0,)))
        m = srev >= (target - acc)
        has = _scalar0(plsc.all_reduce_population_count(m)) > 0
        j0 = _scalar0(plsc.all_reduce_ffs(m))
        bl = (L - 1) - j0
        s_j0 = _extract(srev, j0)
        h_bl = _extract(h, bl)
        tot = _extract(srev, L - 1)
        b_new = v * L + bl
        ca_new = acc + s_j0 - h_bl
        return (
            v - 1,
            jnp.where(has, acc, acc + tot),
            has,
            jnp.where(has, b_new, b),
            jnp.where(has, ca_new, ca),
        )

    _, _, _, b, ca = lax.while_loop(
        cond, body, (v_start, acc0, False, jnp.int32(0), jnp.int32(0))
    )
    return b, ca


def _body(logits_hbm, probs_out, idx_out, data, hist, cvals, cidx,
          selx, oprob, oidx, sem0, sem1, sem2, sem3):
    nc = plsc.get_sparse_core_info().num_cores
    wid = lax.axis_index("s") * nc + lax.axis_index("c")

    row_copy = pltpu.async_copy(logits_hbm.at[wid], data, sem0)
    _zero_hist(hist, NBKT1 // L)
    row_copy.wait()

    def p1_body(i, mx):
        x = data[pl.ds(i * L, L)]
        key = _key16(x)
        bkt = (key >> 19) + jnp.int32(NBKT1 // 2)
        plsc.addupdate_scatter(hist, [bkt], jnp.ones((L,), jnp.int32))
        return jnp.maximum(mx, x)

    mx = plsc.parallel_loop(
        0, NCHUNK, unroll=4, carry=jnp.full((L,), NEG_INF, jnp.float32)
    )(p1_body)
    m_val = jnp.max(mx)
    m_y = jnp.full((L,), m_val) / jnp.full((L,), TEMP, jnp.float32)

    kmax = jnp.where(
        plsc.bitcast(jnp.full((L,), m_val), jnp.int32) < 0,
        plsc.bitcast(jnp.full((L,), m_val), jnp.int32) ^ jnp.int32(0x7FFFFFFF),
        plsc.bitcast(jnp.full((L,), m_val), jnp.int32),
    )
    kmax0 = _scalar0(kmax)
    v_start1 = ((kmax0 >> 19) + jnp.int32(NBKT1 // 2)) // L
    b1, ca1 = _scan_top(hist, v_start1, jnp.int32(0), jnp.int32(K))
    p1 = b1 - jnp.int32(NBKT1 // 2)
    n1 = ca1 + _extract(hist[pl.ds((b1 // L) * L, L)], b1 % L)

    def level23(_):
        _zero_hist(hist, NBKT2 // L)

        def h2_body(i, _):
            x = data[pl.ds(i * L, L)]
            key = _key16(x)
            sel = (key >> 19) == p1
            bkt = (key >> 8) & jnp.int32(0x7FF)
            plsc.addupdate_scatter(
                hist, [bkt], jnp.ones((L,), jnp.int32), mask=sel
            )
            return 0

        lax.fori_loop(0, NCHUNK, h2_body, 0)
        b2, ca2 = _scan_top(hist, jnp.int32(NBKT2 // L - 1), ca1, jnp.int32(K))
        n2 = ca2 + _extract(hist[pl.ds((b2 // L) * L, L)], b2 % L)
        t2 = (p1 << 19) | (b2 << 8)

        def level3(_):
            _zero_hist(hist, NBKT3 // L)
            pfx2 = (p1 << 11) | b2

            def h3_body(i, _):
                x = data[pl.ds(i * L, L)]
                key = _key16(x)
                sel = (key >> 8) == pfx2
                bkt = key & jnp.int32(0xFF)
                plsc.addupdate_scatter(
                    hist, [bkt], jnp.ones((L,), jnp.int32), mask=sel
                )
                return 0

            lax.fori_loop(0, NCHUNK, h3_body, 0)
            b3, ca3 = _scan_top(
                hist, jnp.int32(NBKT3 // L - 1), ca2, jnp.int32(K)
            )
            t_eq = t2 | b3
            return t_eq + jnp.int32(1), jnp.bool_(True), t_eq

        return lax.cond(
            n2 <= CAP,
            lambda _: (t2, jnp.bool_(False), jnp.int32(0)),
            level3,
            0,
        )

    thresh, need_eq, t_eq = lax.cond(
        n1 <= CAP,
        lambda _: (p1 << 19, jnp.bool_(False), jnp.int32(0)),
        level23,
        0,
    )

    tempv = jnp.full((L,), TEMP, jnp.float32)

    def p2_body(i, carry):
        ssum, cntv = carry
        x = data[pl.ds(i * L, L)]
        ssum = ssum + jnp.exp(x / tempv - m_y)
        key = _key16(x)
        m = key >= thresh
        inc = plsc.cumsum(jnp.where(m, jnp.int32(1), jnp.int32(0)))
        pos = cntv + inc - 1
        plsc.store_scatter(cvals, [pos], x, mask=m)
        plsc.store_scatter(cidx, [pos], i * L + _iota(), mask=m)
        cntv = cntv + plsc.all_reduce_population_count(m)
        return ssum, cntv

    ssum, cntv = plsc.parallel_loop(
        0, NCHUNK, unroll=4,
        carry=(jnp.zeros((L,), jnp.float32), jnp.zeros((L,), jnp.int32)),
    )(p2_body)
    s_val = jnp.sum(ssum)
    cnt = _scalar0(cntv)

    def eq_phase(cnt0):
        def cond(st):
            i, cnt = st
            return jnp.logical_and(i < NCHUNK, cnt < K)

        def body(st):
            i, cnt = st
            x = data[pl.ds(i * L, L)]
            key = _key16(x)
            m = key == t_eq
            inc = plsc.cumsum(jnp.where(m, jnp.int32(1), jnp.int32(0)))
            pos = cnt + inc - 1
            plsc.store_scatter(cvals, [pos], x, mask=m)
            plsc.store_scatter(cidx, [pos], i * L + _iota(), mask=m)
            cnt = cnt + _scalar0(plsc.all_reduce_population_count(m))
            return i + 1, cnt

        _, cnt1 = lax.while_loop(cond, body, (jnp.int32(0), cnt0))
        return cnt1

    cnt = lax.cond(need_eq, eq_phase, lambda c: c, cnt)

    nv = (cnt + L - 1) // L

    def pad_body(j, _):
        v = cvals[pl.ds(j * L, L)]
        posn = j * L + _iota()
        cvals[pl.ds(j * L, L)] = jnp.where(
            posn >= cnt, jnp.full((L,), NEG_INF, jnp.float32), v
        )
        return 0

    lax.fori_loop(0, nv, pad_body, 0)

    lane0 = _iota() == 0

    def sel_body(k, _):
        def scan_body(j, st):
            mvec, jvec = st
            v = cvals[pl.ds(j * L, L)]
            gt = v > mvec
            return jnp.where(gt, v, mvec), jnp.where(gt, j, jvec)

        mvec, jvec = lax.fori_loop(
            0, nv, scan_body,
            (jnp.full((L,), NEG_INF, jnp.float32), jnp.zeros((L,), jnp.int32)),
        )
        mval = jnp.max(mvec)
        posn = jvec * L + _iota()
        pos = jnp.min(
            jnp.where(mvec == mval, posn, jnp.int32(2 ** 30))
        )
        posv = jnp.zeros((L,), jnp.int32) + pos
        gidx = plsc.load_gather(cidx, [posv])
        plsc.store_scatter(
            cvals, [posv], jnp.full((L,), NEG_INF, jnp.float32), mask=lane0
        )
        kv = jnp.zeros((L,), jnp.int32) + k
        plsc.store_scatter(selx, [kv], jnp.full((L,), 0.0) + mval, mask=lane0)
        plsc.store_scatter(oidx, [kv], gidx, mask=lane0)
        return 0

    lax.fori_loop(0, K, sel_body, 0)

    carry = jnp.float32(0.0)
    psum = jnp.zeros((L,), jnp.float32)
    s_vec = jnp.zeros((L,), jnp.float32) + s_val
    for t in range(K // L):
        xv = selx[pl.ds(t * L, L)]
        p = jnp.exp(xv / tempv - m_y) / s_vec
        c = plsc.cumsum(p) + carry
        carry = _extract(c, L - 1)
        p = jnp.where((c - p) > TOP_P, jnp.float32(0.0), p)
        psum = psum + p
        oprob[pl.ds(t * L, L)] = p
    tot_vec = jnp.zeros((L,), jnp.float32) + jnp.sum(psum)
    for t in range(K // L):
        oprob[pl.ds(t * L, L)] = oprob[pl.ds(t * L, L)] / tot_vec

    pltpu.sync_copy(oprob, probs_out.at[pl.ds(wid * K, K)])
    pltpu.sync_copy(oidx, idx_out.at[pl.ds(wid * K, K)])


@jax.jit
def _sc_topk_sample(x2d):
    mesh = plsc.VectorSubcoreMesh(core_axis_name="c", subcore_axis_name="s")
    fn = pl.kernel(
        _body,
        out_type=[
            jax.ShapeDtypeStruct((B * K,), jnp.float32),
            jax.ShapeDtypeStruct((B * K,), jnp.int32),
        ],
        mesh=mesh,
        scratch_types=[
            pltpu.VMEM((V,), jnp.float32),
            pltpu.VMEM((NBKT1,), jnp.int32),
            pltpu.VMEM((CAP,), jnp.float32),
            pltpu.VMEM((CAP,), jnp.int32),
            pltpu.VMEM((K,), jnp.float32),
            pltpu.VMEM((K,), jnp.float32),
            pltpu.VMEM((K,), jnp.int32),
            pltpu.SemaphoreType.DMA,
            pltpu.SemaphoreType.DMA,
            pltpu.SemaphoreType.DMA,
            pltpu.SemaphoreType.DMA,
        ],
        name="sc_topk_sampler",
        compiler_params=pltpu.CompilerParams(
            needs_layout_passes=False, use_tc_tiling_on_sc=True
        ),
    )
    pr, ix = fn(x2d)
    return pr.reshape(B, K), ix.reshape(B, K)


def kernel(logits):
    x = logits[:, -1]
    probs_sorted, indices = _sc_topk_sample(x)
    skey = jax.random.fold_in(jax.random.key(42), 0)
    next_token = jax.random.categorical(skey, logits=jnp.log(probs_sorted))
    next_token = jnp.take_along_axis(indices, next_token[..., None], axis=-1)
    next_token = jnp.squeeze(next_token, axis=-1)
    return next_token, probs_sorted


# speedup vs baseline: 1.0643x; 1.0127x over previous
import functools

import jax
import jax.numpy as jnp
from jax import lax
from jax.experimental import pallas as pl
from jax.experimental.pallas import tpu as pltpu
from jax.experimental.pallas import tpu_sc as plsc

B = 32
V = 100000
K = 64
L = 16
NCHUNK = V // L
NBKT1 = 8192
NBKT2 = 2048
NBKT3 = 256
CAP = 512
TEMP = 0.7
TOP_P = 0.95
NEG_INF = float("-inf")


def _iota():
    return lax.iota(jnp.int32, L)


def _extract(vec, j):
    return jnp.sum(jnp.where(_iota() == j, vec, jnp.zeros_like(vec)))


def _scalar0(vec):
    return _extract(vec, 0)


def _key16(x):
    bits = plsc.bitcast(x, jnp.int32)
    return jnp.where(bits < 0, bits ^ jnp.int32(0x7FFFFFFF), bits)


def _zero_hist(hist, nvregs):
    def zbody(j):
        hist[pl.ds(j * L, L)] = jnp.zeros((L,), jnp.int32)

    plsc.parallel_loop(0, nvregs, unroll=8)(zbody)


def _scan_top(hist, v_start, acc0, target):
    def cond(st):
        v, _, found, _, _ = st
        return jnp.logical_and(jnp.logical_not(found), v >= 0)

    def body(st):
        v, acc, _, b, ca = st
        h = hist[pl.ds(v * L, L)]
        srev = plsc.cumsum(lax.rev(h, (0,)))
        m = srev >= (target - acc)
        has = _scalar0(plsc.all_reduce_population_count(m)) > 0
        j0 = _scalar0(plsc.all_reduce_ffs(m))
        bl = (L - 1) - j0
        s_j0 = _extract(srev, j0)
        h_bl = _extract(h, bl)
        tot = _extract(srev, L - 1)
        b_new = v * L + bl
        ca_new = acc + s_j0 - h_bl
        return (
            v - 1,
            jnp.where(has, acc, acc + tot),
            has,
            jnp.where(has, b_new, b),
            jnp.where(has, ca_new, ca),
        )

    _, _, _, b, ca = lax.while_loop(
        cond, body, (v_start, acc0, False, jnp.int32(0), jnp.int32(0))
    )
    return b, ca


def _body(logits_hbm, probs_out, idx_out, data, hist, cvals, cidx,
          selx, oprob, oidx, sem0, sem1, sem2, sem3):
    nc = plsc.get_sparse_core_info().num_cores
    wid = lax.axis_index("s") * nc + lax.axis_index("c")

    row_copy = pltpu.async_copy(logits_hbm.at[wid], data, sem0)
    _zero_hist(hist, NBKT1 // L)
    row_copy.wait()

    def p1_body(i, mx):
        x = data[pl.ds(i * L, L)]
        key = _key16(x)
        bkt = (key >> 19) + jnp.int32(NBKT1 // 2)
        plsc.addupdate_scatter(hist, [bkt], jnp.ones((L,), jnp.int32))
        return jnp.maximum(mx, x)

    mx = plsc.parallel_loop(
        0, NCHUNK, unroll=8, carry=jnp.full((L,), NEG_INF, jnp.float32)
    )(p1_body)
    m_val = jnp.max(mx)
    m_y = jnp.full((L,), m_val) / jnp.full((L,), TEMP, jnp.float32)

    kmax = jnp.where(
        plsc.bitcast(jnp.full((L,), m_val), jnp.int32) < 0,
        plsc.bitcast(jnp.full((L,), m_val), jnp.int32) ^ jnp.int32(0x7FFFFFFF),
        plsc.bitcast(jnp.full((L,), m_val), jnp.int32),
    )
    kmax0 = _scalar0(kmax)
    v_start1 = ((kmax0 >> 19) + jnp.int32(NBKT1 // 2)) // L
    b1, ca1 = _scan_top(hist, v_start1, jnp.int32(0), jnp.int32(K))
    p1 = b1 - jnp.int32(NBKT1 // 2)
    n1 = ca1 + _extract(hist[pl.ds((b1 // L) * L, L)], b1 % L)

    def level23(_):
        _zero_hist(hist, NBKT2 // L)

        def h2_body(i, _):
            x = data[pl.ds(i * L, L)]
            key = _key16(x)
            sel = (key >> 19) == p1
            bkt = (key >> 8) & jnp.int32(0x7FF)
            plsc.addupdate_scatter(
                hist, [bkt], jnp.ones((L,), jnp.int32), mask=sel
            )
            return 0

        lax.fori_loop(0, NCHUNK, h2_body, 0)
        b2, ca2 = _scan_top(hist, jnp.int32(NBKT2 // L - 1), ca1, jnp.int32(K))
        n2 = ca2 + _extract(hist[pl.ds((b2 // L) * L, L)], b2 % L)
        t2 = (p1 << 19) | (b2 << 8)

        def level3(_):
            _zero_hist(hist, NBKT3 // L)
            pfx2 = (p1 << 11) | b2

            def h3_body(i, _):
                x = data[pl.ds(i * L, L)]
                key = _key16(x)
                sel = (key >> 8) == pfx2
                bkt = key & jnp.int32(0xFF)
                plsc.addupdate_scatter(
                    hist, [bkt], jnp.ones((L,), jnp.int32), mask=sel
                )
                return 0

            lax.fori_loop(0, NCHUNK, h3_body, 0)
            b3, ca3 = _scan_top(
                hist, jnp.int32(NBKT3 // L - 1), ca2, jnp.int32(K)
            )
            t_eq = t2 | b3
            return t_eq + jnp.int32(1), jnp.bool_(True), t_eq

        return lax.cond(
            n2 <= CAP,
            lambda _: (t2, jnp.bool_(False), jnp.int32(0)),
            level3,
            0,
        )

    thresh, need_eq, t_eq = lax.cond(
        n1 <= CAP,
        lambda _: (p1 << 19, jnp.bool_(False), jnp.int32(0)),
        level23,
        0,
    )

    tempv = jnp.full((L,), TEMP, jnp.float32)

    def p2_body(i, carry):
        ssum, cntv = carry
        x = data[pl.ds(i * L, L)]
        ssum = ssum + jnp.exp(x / tempv - m_y)
        key = _key16(x)
        m = key >= thresh
        inc = plsc.cumsum(jnp.where(m, jnp.int32(1), jnp.int32(0)))
        pos = cntv + inc - 1
        plsc.store_scatter(cvals, [pos], x, mask=m)
        plsc.store_scatter(cidx, [pos], i * L + _iota(), mask=m)
        cntv = cntv + plsc.all_reduce_population_count(m)
        return ssum, cntv

    ssum, cntv = plsc.parallel_loop(
        0, NCHUNK, unroll=8,
        carry=(jnp.zeros((L,), jnp.float32), jnp.zeros((L,), jnp.int32)),
    )(p2_body)
    s_val = jnp.sum(ssum)
    cnt = _scalar0(cntv)

    def eq_phase(cnt0):
        def cond(st):
            i, cnt = st
            return jnp.logical_and(i < NCHUNK, cnt < K)

        def body(st):
            i, cnt = st
            x = data[pl.ds(i * L, L)]
            key = _key16(x)
            m = key == t_eq
            inc = plsc.cumsum(jnp.where(m, jnp.int32(1), jnp.int32(0)))
            pos = cnt + inc - 1
            plsc.store_scatter(cvals, [pos], x, mask=m)
            plsc.store_scatter(cidx, [pos], i * L + _iota(), mask=m)
            cnt = cnt + _scalar0(plsc.all_reduce_population_count(m))
            return i + 1, cnt

        _, cnt1 = lax.while_loop(cond, body, (jnp.int32(0), cnt0))
        return cnt1

    cnt = lax.cond(need_eq, eq_phase, lambda c: c, cnt)

    nv = (cnt + L - 1) // L

    def pad_body(j, _):
        v = cvals[pl.ds(j * L, L)]
        posn = j * L + _iota()
        cvals[pl.ds(j * L, L)] = jnp.where(
            posn >= cnt, jnp.full((L,), NEG_INF, jnp.float32), v
        )
        return 0

    lax.fori_loop(0, nv, pad_body, 0)

    lane0 = _iota() == 0

    def sel_body(k, _):
        def scan_body(j, st):
            mvec, jvec = st
            v = cvals[pl.ds(j * L, L)]
            gt = v > mvec
            return jnp.where(gt, v, mvec), jnp.where(gt, j, jvec)

        mvec, jvec = lax.fori_loop(
            0, nv, scan_body,
            (jnp.full((L,), NEG_INF, jnp.float32), jnp.zeros((L,), jnp.int32)),
        )
        mval = jnp.max(mvec)
        posn = jvec * L + _iota()
        pos = jnp.min(
            jnp.where(mvec == mval, posn, jnp.int32(2 ** 30))
        )
        posv = jnp.zeros((L,), jnp.int32) + pos
        gidx = plsc.load_gather(cidx, [posv])
        plsc.store_scatter(
            cvals, [posv], jnp.full((L,), NEG_INF, jnp.float32), mask=lane0
        )
        kv = jnp.zeros((L,), jnp.int32) + k
        plsc.store_scatter(selx, [kv], jnp.full((L,), 0.0) + mval, mask=lane0)
        plsc.store_scatter(oidx, [kv], gidx, mask=lane0)
        return 0

    lax.fori_loop(0, K, sel_body, 0)

    carry = jnp.float32(0.0)
    psum = jnp.zeros((L,), jnp.float32)
    s_vec = jnp.zeros((L,), jnp.float32) + s_val
    for t in range(K // L):
        xv = selx[pl.ds(t * L, L)]
        p = jnp.exp(xv / tempv - m_y) / s_vec
        c = plsc.cumsum(p) + carry
        carry = _extract(c, L - 1)
        p = jnp.where((c - p) > TOP_P, jnp.float32(0.0), p)
        psum = psum + p
        oprob[pl.ds(t * L, L)] = p
    tot_vec = jnp.zeros((L,), jnp.float32) + jnp.sum(psum)
    for t in range(K // L):
        oprob[pl.ds(t * L, L)] = oprob[pl.ds(t * L, L)] / tot_vec

    pltpu.sync_copy(oprob, probs_out.at[pl.ds(wid * K, K)])
    pltpu.sync_copy(oidx, idx_out.at[pl.ds(wid * K, K)])


@jax.jit
def _sc_topk_sample(x2d):
    mesh = plsc.VectorSubcoreMesh(core_axis_name="c", subcore_axis_name="s")
    fn = pl.kernel(
        _body,
        out_type=[
            jax.ShapeDtypeStruct((B * K,), jnp.float32),
            jax.ShapeDtypeStruct((B * K,), jnp.int32),
        ],
        mesh=mesh,
        scratch_types=[
            pltpu.VMEM((V,), jnp.float32),
            pltpu.VMEM((NBKT1,), jnp.int32),
            pltpu.VMEM((CAP,), jnp.float32),
            pltpu.VMEM((CAP,), jnp.int32),
            pltpu.VMEM((K,), jnp.float32),
            pltpu.VMEM((K,), jnp.float32),
            pltpu.VMEM((K,), jnp.int32),
            pltpu.SemaphoreType.DMA,
            pltpu.SemaphoreType.DMA,
            pltpu.SemaphoreType.DMA,
            pltpu.SemaphoreType.DMA,
        ],
        name="sc_topk_sampler",
        compiler_params=pltpu.CompilerParams(
            needs_layout_passes=False, use_tc_tiling_on_sc=True
        ),
    )
    pr, ix = fn(x2d)
    return pr.reshape(B, K), ix.reshape(B, K)


def kernel(logits):
    x = logits[:, -1]
    probs_sorted, indices = _sc_topk_sample(x)
    skey = jax.random.fold_in(jax.random.key(42), 0)
    next_token = jax.random.categorical(skey, logits=jnp.log(probs_sorted))
    next_token = jnp.take_along_axis(indices, next_token[..., None], axis=-1)
    next_token = jnp.squeeze(next_token, axis=-1)
    return next_token, probs_sorted
